# Initial kernel scaffold; baseline (speedup 1.0000x reference)
import jax
import jax.numpy as jnp
from jax.experimental import pallas as pl


def kernel(img_msb, img_lsb, img_org, feature_msb, feature_lsb, lut_cat, s_layers, w_layers, luts):
    raise NotImplementedError("write your pallas kernel here")


# speedup vs baseline: 21.8068x; 21.8068x over previous
import functools

import numpy as np
import jax
import jax.numpy as jnp
from jax import lax
from jax.experimental import pallas as pl
from jax.experimental.pallas import tpu as pltpu
from jax.experimental.pallas import tpu_sc as plsc

_DIM = 33
_D2 = _DIM * _DIM
_D3 = _DIM * _D2
_FEAT_T = 16 * 17 * 17 * 15 + 16 * 17 * 15 + 16 * 15 + 1
_P = 512 * 512
_NW = 32
_LANES = 16

_col = np.arange(300)
_n_of = (_col % 60) // 3
_c_of = (_col % 60) % 3
_s_of = _col // 60
_E_NP = (np.arange(20)[:, None] == _n_of[None, :]).astype(np.float32)
_row = np.arange(15)
_MASK_NP = ((_s_of[None, :] == (_row % 5)[:, None])
            & (_c_of[None, :] == (_row // 5)[:, None])).astype(np.float32)


def _sc_mesh():
    return plsc.VectorSubcoreMesh(core_axis_name="c", subcore_axis_name="s")


_IDX_N = 2 * _P
_PER_W1 = _IDX_N // _NW
_CH1 = 128
_NCH1 = _PER_W1 // _CH1


def _feat_sum_body(tab_hbm, idx_hbm, out_hbm, idx_v, rows_v, acc_v, sem):
    wid = lax.axis_index("s") * 2 + lax.axis_index("c")
    base = wid * _PER_W1
    zero = jnp.zeros((_LANES,), jnp.float32)

    def chunk(ci, carry):
        a0, a1 = carry
        pltpu.sync_copy(idx_hbm.at[pl.ds(base + ci * _CH1, _CH1)], idx_v)
        pltpu.async_copy(tab_hbm.at[idx_v], rows_v, sem).wait()

        def row(i, cc):
            c0, c1 = cc
            return (c0 + rows_v[i, pl.ds(0, 16)],
                    c1 + rows_v[i, pl.ds(16, 16)])

        c0, c1 = lax.fori_loop(0, _CH1, row, (zero, zero))
        return (a0 + c0, a1 + c1)

    a0, a1 = lax.fori_loop(0, _NCH1, chunk, (zero, zero))
    acc_v[pl.ds(0, 16)] = a0
    acc_v[pl.ds(16, 16)] = a1
    pltpu.sync_copy(acc_v, out_hbm.at[wid])


def _feat_sum(tab, idx_all):
    k = pl.kernel(
        _feat_sum_body,
        mesh=_sc_mesh(),
        compiler_params=pltpu.CompilerParams(use_tc_tiling_on_sc=False),
        out_type=jax.ShapeDtypeStruct((_NW, 32), jnp.float32),
        scratch_types=[
            pltpu.VMEM((_CH1,), jnp.int32),
            pltpu.VMEM((_CH1, 32), jnp.float32),
            pltpu.VMEM((32,), jnp.float32),
            pltpu.SemaphoreType.DMA,
        ],
    )
    return k(tab, idx_all)


def _recon_body(w_ref, e_ref, m_ref, luts_ref, wl_ref, sl_ref, o_ref):
    wb = jnp.dot(w_ref[...], e_ref[...], preferred_element_type=jnp.float32)
    w2 = m_ref[...] * wb
    b1 = jnp.dot(w2, luts_ref[...], preferred_element_type=jnp.float32)
    b2 = jnp.dot(b1, wl_ref[...], preferred_element_type=jnp.float32)
    sl = sl_ref[...]
    o_ref[...] = jnp.concatenate(
        [jnp.dot(sl, b2[0:5], preferred_element_type=jnp.float32),
         jnp.dot(sl, b2[5:10], preferred_element_type=jnp.float32),
         jnp.dot(sl, b2[10:15], preferred_element_type=jnp.float32)],
        axis=0)


def _recon(w, e, m, luts, w_layers, s_layers):
    return pl.pallas_call(
        _recon_body,
        out_shape=jax.ShapeDtypeStruct((3 * _DIM, _D2), jnp.float32),
    )(w, e, m, luts, w_layers, s_layers)


_PER_W3 = _P // _NW
_CH3 = 512
_NCH3 = _PER_W3 // _CH3
_NV3 = _CH3 // _LANES
_BINSIZE = np.float32(1.000001 / (_DIM - 1))


def _tri_body(lut_hbm, r_hbm, g_hbm, b_hbm, out_hbm,
              lut_v, rch, gch, bch, och, sem):
    wid = lax.axis_index("s") * 2 + lax.axis_index("c")
    base = wid * _PER_W3
    pltpu.sync_copy(lut_hbm, lut_v)
    maxid = jnp.full((_LANES,), _DIM - 2, jnp.int32)
    zeroi = jnp.zeros((_LANES,), jnp.int32)
    one = jnp.float32(1.0)

    def chunk(ci, _):
        off = base + ci * _CH3
        pltpu.sync_copy(r_hbm.at[pl.ds(off, _CH3)], rch)
        pltpu.sync_copy(g_hbm.at[pl.ds(off, _CH3)], gch)
        pltpu.sync_copy(b_hbm.at[pl.ds(off, _CH3)], bch)

        def vec(j, __):
            rv = rch[pl.ds(j * 16, 16)]
            gv = gch[pl.ds(j * 16, 16)]
            bv = bch[pl.ds(j * 16, 16)]
            rs = rv / _BINSIZE
            gs = gv / _BINSIZE
            bs = bv / _BINSIZE
            ri = jnp.minimum(jnp.maximum(rs.astype(jnp.int32), zeroi), maxid)
            gi = jnp.minimum(jnp.maximum(gs.astype(jnp.int32), zeroi), maxid)
            bi = jnp.minimum(jnp.maximum(bs.astype(jnp.int32), zeroi), maxid)
            rd = rs - ri.astype(jnp.float32)
            gd = gs - gi.astype(jnp.float32)
            bd = bs - bi.astype(jnp.float32)
            omr = one - rd
            omg = one - gd
            omb = one - bd
            a0 = jnp.zeros((_LANES,), jnp.float32)
            a1 = jnp.zeros((_LANES,), jnp.float32)
            a2 = jnp.zeros((_LANES,), jnp.float32)
            for db in (0, 1):
                wb_ = bd if db else omb
                bq = (bi + db) * _D2
                b33 = (bi + db) * 33
                for dg in (0, 1):
                    wg_ = (gd if dg else omg) * wb_
                    gq = (gi + dg) * _D2
                    g33 = (gi + dg) * 33
                    for dr in (0, 1):
                        w_ = (rd if dr else omr) * wg_
                        rr = ri + dr
                        i0 = bq + g33 + rr
                        i1 = _D3 + gq + b33 + rr
                        i2 = 2 * _D3 + rr * _D2 + b33 + (gi + dg)
                        a0 = a0 + w_ * plsc.load_gather(lut_v, [i0])
                        a1 = a1 + w_ * plsc.load_gather(lut_v, [i1])
                        a2 = a2 + w_ * plsc.load_gather(lut_v, [i2])
            och[pl.ds(j * 16, 16)] = a0 + rv
            och[pl.ds(_CH3 + j * 16, 16)] = a1 + gv
            och[pl.ds(2 * _CH3 + j * 16, 16)] = a2 + bv
            return 0

        lax.fori_loop(0, _NV3, vec, 0)
        pltpu.sync_copy(och.at[pl.ds(0, _CH3)],
                        out_hbm.at[pl.ds(off, _CH3)])
        pltpu.sync_copy(och.at[pl.ds(_CH3, _CH3)],
                        out_hbm.at[pl.ds(_P + off, _CH3)])
        pltpu.sync_copy(och.at[pl.ds(2 * _CH3, _CH3)],
                        out_hbm.at[pl.ds(2 * _P + off, _CH3)])
        return 0

    lax.fori_loop(0, _NCH3, chunk, 0)


def _trilinear(lut_flat, r, g, b):
    k = pl.kernel(
        _tri_body,
        mesh=_sc_mesh(),
        compiler_params=pltpu.CompilerParams(needs_layout_passes=False),
        out_type=jax.ShapeDtypeStruct((3 * _P,), jnp.float32),
        scratch_types=[
            pltpu.VMEM((3 * _D3,), jnp.float32),
            pltpu.VMEM((_CH3,), jnp.float32),
            pltpu.VMEM((_CH3,), jnp.float32),
            pltpu.VMEM((_CH3,), jnp.float32),
            pltpu.VMEM((3 * _CH3,), jnp.float32),
            pltpu.SemaphoreType.DMA,
        ],
    )
    return k(lut_flat, r, g, b)


@jax.jit
def _run(img_msb, img_lsb, img_org, feature_msb, feature_lsb, lut_cat,
         s_layers, w_layers, luts):
    wvec = jnp.array([16 * 17 * 17, 16 * 17, 16], jnp.float32).reshape(1, 3, 1, 1)
    idx_m = jnp.sum(img_msb.astype(jnp.float32) * wvec, axis=1
                    ).reshape(-1).astype(jnp.int32)
    idx_l = jnp.sum(img_lsb.astype(jnp.float32) * wvec, axis=1
                    ).reshape(-1).astype(jnp.int32) + _FEAT_T
    idx_all = jnp.concatenate([idx_m, idx_l])
    tab = jnp.pad(jnp.concatenate([feature_msb, feature_lsb], axis=0),
                  ((0, 0), (0, 12)))
    sums = _feat_sum(tab, idx_all)

    mid = jnp.sum(sums, axis=0)[:20] / np.float32(_P)
    mid = jnp.clip(jnp.round(mid * 4.0) / 4.0, -32.0, 31.75)
    midq = ((mid * 4.0).astype(jnp.int32) + 128).reshape(10, 2)
    index = midq[:, 0] * 256 + midq[:, 1]
    weights = jnp.sum(lut_cat[jnp.arange(10), index], axis=0).reshape(1, 20)

    comb = _recon(weights, jnp.asarray(_E_NP), jnp.asarray(_MASK_NP),
                  luts, w_layers, s_layers)
    comb3 = comb.reshape(3, _DIM, _DIM, _DIM)
    d3lut = jnp.stack([comb3[0],
                       comb3[1].transpose(1, 0, 2),
                       comb3[2].transpose(1, 2, 0)])[None]

    x = img_org.reshape(3, _P)
    res = _trilinear(comb.reshape(-1), x[0], x[1], x[2])
    return res.reshape(1, 3, 512, 512), d3lut


def kernel(img_msb, img_lsb, img_org, feature_msb, feature_lsb, lut_cat,
           s_layers, w_layers, luts):
    return _run(img_msb, img_lsb, img_org, feature_msb, feature_lsb,
                lut_cat, s_layers, w_layers, luts)
